# baseline (speedup 1.0000x reference)
import functools

import jax
import jax.numpy as jnp
from jax import lax
from jax.experimental import pallas as pl
from jax.experimental.pallas import tpu as pltpu
from jax.experimental.pallas import tpu_sc as plsc

B, S, D = 4096, 200, 64
F = S * D
NC, NS = 2, 16
NW = NC * NS
BPW = B // NW
C = 1
CH = C * F
NBUF = 4
CHUNKS = BPW // C
G = CHUNKS // NBUF
U = 8
LANES = 16

_mesh = plsc.VectorSubcoreMesh(
    core_axis_name="c", subcore_axis_name="s", num_cores=NC, num_subcores=NS
)


@functools.partial(
    pl.kernel,
    out_type=jax.ShapeDtypeStruct((B * F,), jnp.float32),
    mesh=_mesh,
    scratch_types=(
        [pltpu.VMEM((F,), jnp.float32)]
        + [pltpu.VMEM((CH,), jnp.float32) for _ in range(NBUF)]
        + [pltpu.SemaphoreType.DMA for _ in range(2 * NBUF)]
    ),
)
def _pos_add(x_hbm, emb_hbm, out_hbm, emb_v, *rest):
    bufs = rest[:NBUF]
    in_sems = rest[NBUF : 2 * NBUF]
    out_sems = rest[2 * NBUF :]

    wid = lax.axis_index("s") * NC + lax.axis_index("c")
    base = wid * (BPW * F)

    pltpu.sync_copy(emb_hbm, emb_v)

    def start_in(k, ci):
        pltpu.async_copy(x_hbm.at[pl.ds(base + ci * CH, CH)], bufs[k], in_sems[k])

    def wait_in(k):
        pltpu.make_async_copy(x_hbm.at[pl.ds(0, CH)], bufs[k], in_sems[k]).wait()

    def start_out(k, ci):
        pltpu.async_copy(bufs[k], out_hbm.at[pl.ds(base + ci * CH, CH)], out_sems[k])

    def wait_out(k):
        pltpu.make_async_copy(bufs[k], out_hbm.at[pl.ds(0, CH)], out_sems[k]).wait()

    def add_table(k):
        buf = bufs[k]
        for c in range(C):
            coff = c * F

            def body(t, _):
                for u in range(U):
                    o = t * (LANES * U) + u * LANES
                    buf[pl.ds(coff + o, LANES)] = (
                        buf[pl.ds(coff + o, LANES)] + emb_v[pl.ds(o, LANES)]
                    )
                return 0

            lax.fori_loop(0, F // (LANES * U), body, 0)

    for k in range(NBUF):
        start_in(k, k)

    def group(g, _):
        for k in range(NBUF):
            ci = g * NBUF + k
            wait_in(k)
            add_table(k)
            start_out(k, ci)
            wait_out(k)
            start_in(k, ci + NBUF)
        return 0

    lax.fori_loop(0, G - 1, group, 0)

    for k in range(NBUF):
        wait_in(k)
        add_table(k)
        start_out(k, (G - 1) * NBUF + k)
    for k in range(NBUF):
        wait_out(k)


def kernel(x, emb):
    out = _pos_add(x.reshape(-1), emb.reshape(-1))
    return out.reshape(B, S, D)


# speedup vs baseline: 1.3086x; 1.3086x over previous
import functools

import jax
import jax.numpy as jnp
from jax import lax
from jax.experimental import pallas as pl
from jax.experimental.pallas import tpu as pltpu
from jax.experimental.pallas import tpu_sc as plsc

B, S, D = 4096, 200, 64
NC, NS = 2, 16
NW = NC * NS
BPW = B // NW
NBUF = 4
G = BPW // NBUF
LANES = 16
DV = D // LANES

_mesh = plsc.VectorSubcoreMesh(
    core_axis_name="c", subcore_axis_name="s", num_cores=NC, num_subcores=NS
)


@functools.partial(
    pl.kernel,
    out_type=jax.ShapeDtypeStruct((B, S, D), jnp.float32),
    mesh=_mesh,
    compiler_params=pltpu.CompilerParams(use_tc_tiling_on_sc=True),
    scratch_types=(
        [pltpu.VMEM((S, D), jnp.float32)]
        + [pltpu.VMEM((S, D), jnp.float32) for _ in range(NBUF)]
        + [pltpu.SemaphoreType.DMA for _ in range(2 * NBUF)]
    ),
)
def _pos_add(x_hbm, emb_hbm, out_hbm, emb_v, *rest):
    bufs = rest[:NBUF]
    in_sems = rest[NBUF : 2 * NBUF]
    out_sems = rest[2 * NBUF :]

    wid = lax.axis_index("s") * NC + lax.axis_index("c")
    base = wid * BPW

    pltpu.sync_copy(emb_hbm, emb_v)

    def start_in(k, b):
        pltpu.async_copy(x_hbm.at[base + b], bufs[k], in_sems[k])

    def wait_in(k):
        pltpu.make_async_copy(x_hbm.at[0], bufs[k], in_sems[k]).wait()

    def start_out(k, b):
        pltpu.async_copy(bufs[k], out_hbm.at[base + b], out_sems[k])

    def wait_out(k):
        pltpu.make_async_copy(bufs[k], out_hbm.at[0], out_sems[k]).wait()

    def add_table(k):
        buf = bufs[k]

        def body(s, _):
            for u in range(DV):
                sl = pl.ds(u * LANES, LANES)
                buf[s, sl] = buf[s, sl] + emb_v[s, sl]
            return 0

        lax.fori_loop(0, S, body, 0)

    for k in range(NBUF):
        start_in(k, k)

    def group(g, _):
        for k in range(NBUF):
            b = g * NBUF + k
            wait_in(k)
            add_table(k)
            start_out(k, b)
            wait_out(k)
            start_in(k, b + NBUF)
        return 0

    lax.fori_loop(0, G - 1, group, 0)

    for k in range(NBUF):
        wait_in(k)
        add_table(k)
        start_out(k, (G - 1) * NBUF + k)
    for k in range(NBUF):
        wait_out(k)


def kernel(x, emb):
    return _pos_add(x, emb)


# speedup vs baseline: 1.3117x; 1.0024x over previous
import functools

import jax
import jax.numpy as jnp
from jax import lax
from jax.experimental import pallas as pl
from jax.experimental.pallas import tpu as pltpu
from jax.experimental.pallas import tpu_sc as plsc

B, S, D = 4096, 200, 64
NC, NS = 2, 16
NW = NC * NS
BPW = B // NW
NBUF = 2
G = BPW // NBUF
LANES = 16
DV = D // LANES

_mesh = plsc.VectorSubcoreMesh(
    core_axis_name="c", subcore_axis_name="s", num_cores=NC, num_subcores=NS
)


@functools.partial(
    pl.kernel,
    out_type=jax.ShapeDtypeStruct((B, S, D), jnp.float32),
    mesh=_mesh,
    compiler_params=pltpu.CompilerParams(use_tc_tiling_on_sc=True),
    scratch_types=(
        [pltpu.VMEM((S, D), jnp.float32)]
        + [pltpu.VMEM((S, D), jnp.float32) for _ in range(2 * NBUF)]
        + [pltpu.SemaphoreType.DMA for _ in range(2 * NBUF)]
    ),
)
def _pos_add(x_hbm, emb_hbm, out_hbm, emb_v, *rest):
    in_bufs = rest[:NBUF]
    out_bufs = rest[NBUF : 2 * NBUF]
    in_sems = rest[2 * NBUF : 3 * NBUF]
    out_sems = rest[3 * NBUF :]

    wid = lax.axis_index("s") * NC + lax.axis_index("c")
    base = wid * BPW

    pltpu.sync_copy(emb_hbm, emb_v)

    def start_in(k, b):
        pltpu.async_copy(x_hbm.at[base + b], in_bufs[k], in_sems[k])

    def wait_in(k):
        pltpu.make_async_copy(x_hbm.at[0], in_bufs[k], in_sems[k]).wait()

    def start_out(k, b):
        pltpu.async_copy(out_bufs[k], out_hbm.at[base + b], out_sems[k])

    def wait_out(k):
        pltpu.make_async_copy(out_bufs[k], out_hbm.at[0], out_sems[k]).wait()

    def add_table(k):
        src = in_bufs[k]
        dst = out_bufs[k]

        def body(s, _):
            for u in range(DV):
                sl = pl.ds(u * LANES, LANES)
                dst[s, sl] = src[s, sl] + emb_v[s, sl]
            return 0

        lax.fori_loop(0, S, body, 0)

    for k in range(NBUF):
        start_in(k, k)

    for k in range(NBUF):
        wait_in(k)
        add_table(k)
        start_out(k, k)
        start_in(k, NBUF + k)

    def group(g, _):
        for k in range(NBUF):
            b = g * NBUF + k
            wait_in(k)
            wait_out(k)
            add_table(k)
            start_out(k, b)
            start_in(k, b + NBUF)
        return 0

    lax.fori_loop(1, G - 1, group, 0)

    for k in range(NBUF):
        b = (G - 1) * NBUF + k
        wait_in(k)
        wait_out(k)
        add_table(k)
        start_out(k, b)
    for k in range(NBUF):
        wait_out(k)


def kernel(x, emb):
    return _pos_add(x, emb)


# speedup vs baseline: 6.5335x; 4.9811x over previous
import functools

import jax
import jax.numpy as jnp
from jax import lax
from jax.experimental import pallas as pl
from jax.experimental.pallas import tpu as pltpu
from jax.experimental.pallas import tpu_sc as plsc

B, S, D = 4096, 200, 64
NC, NS = 2, 16
NW = NC * NS
R = S * D
DCH = 4
SPS = D // DCH
SPW = (S * SPS) // NW
RPW = SPW * DCH
NBUF = 2
G = SPW // NBUF
LANES = 16
NSL = B // LANES

_mesh = plsc.VectorSubcoreMesh(
    core_axis_name="c", subcore_axis_name="s", num_cores=NC, num_subcores=NS
)


@functools.partial(
    pl.kernel,
    out_type=jax.ShapeDtypeStruct((S, D, B), jnp.float32),
    mesh=_mesh,
    compiler_params=pltpu.CompilerParams(use_tc_tiling_on_sc=True),
    scratch_types=(
        [pltpu.VMEM((RPW, LANES), jnp.float32)]
        + [pltpu.VMEM((DCH, B), jnp.float32) for _ in range(2 * NBUF)]
        + [pltpu.SemaphoreType.DMA for _ in range(2 * NBUF)]
    ),
)
def _pos_add_t(xt_hbm, ex_hbm, out_hbm, emb_v, *rest):
    in_bufs = rest[:NBUF]
    out_bufs = rest[NBUF : 2 * NBUF]
    in_sems = rest[2 * NBUF : 3 * NBUF]
    out_sems = rest[3 * NBUF :]

    wid = lax.axis_index("s") * NC + lax.axis_index("c")
    base = wid * SPW

    pltpu.sync_copy(ex_hbm.at[pl.ds(base * DCH, RPW)], emb_v)

    def coords(i):
        slab = base + i
        return slab // SPS, (slab % SPS) * DCH

    def start_in(k, i):
        s, d0 = coords(i)
        pltpu.async_copy(xt_hbm.at[s, pl.ds(d0, DCH)], in_bufs[k], in_sems[k])

    def wait_in(k):
        pltpu.make_async_copy(
            xt_hbm.at[0, pl.ds(0, DCH)], in_bufs[k], in_sems[k]
        ).wait()

    def start_out(k, i):
        s, d0 = coords(i)
        pltpu.async_copy(out_bufs[k], out_hbm.at[s, pl.ds(d0, DCH)], out_sems[k])

    def wait_out(k):
        pltpu.make_async_copy(
            out_bufs[k], out_hbm.at[0, pl.ds(0, DCH)], out_sems[k]
        ).wait()

    def add_slab(k, i):
        src = in_bufs[k]
        dst = out_bufs[k]
        vecs = [emb_v[i * DCH + j] for j in range(DCH)]

        def body(u, _):
            sl = pl.ds(u * LANES, LANES)
            for j in range(DCH):
                dst[j, sl] = src[j, sl] + vecs[j]
            return 0

        lax.fori_loop(0, NSL, body, 0)

    for k in range(NBUF):
        start_in(k, k)

    for k in range(NBUF):
        wait_in(k)
        add_slab(k, k)
        start_out(k, k)
        start_in(k, NBUF + k)

    def group(g, _):
        for k in range(NBUF):
            i = g * NBUF + k
            wait_in(k)
            wait_out(k)
            add_slab(k, i)
            start_out(k, i)
            start_in(k, i + NBUF)
        return 0

    lax.fori_loop(1, G - 1, group, 0)

    for k in range(NBUF):
        i = (G - 1) * NBUF + k
        wait_in(k)
        wait_out(k)
        add_slab(k, i)
        start_out(k, i)
    for k in range(NBUF):
        wait_out(k)


def kernel(x, emb):
    xt = jnp.transpose(x, (1, 2, 0))
    ex = jnp.broadcast_to(jnp.reshape(emb, (R, 1)), (R, LANES))
    out_t = _pos_add_t(xt, ex)
    return jnp.transpose(out_t, (2, 0, 1))
